# baseline (speedup 1.0000x reference)
import jax
import jax.numpy as jnp
from jax import lax
from jax.experimental import pallas as pl
from jax.experimental.pallas import tpu as pltpu

_E, _L, _N, _F = 4, 5, 512, 128
_S = 1000
_W = 25
_NSTEP = _S // _W
_JT = 32
_LOG2E = 1.4426950408889634
_PAIRS = ((0, 1), (0, 2), (1, 2), (0, 3), (1, 3), (2, 3))


def _prep_kernel(mat_ref, par_ref, proj_ref, scal_ref):
    mat = mat_ref[...]
    mean = jnp.mean(mat, axis=2, keepdims=True)
    var = jnp.sum((mat - mean) ** 2, axis=2) / (_N - 1)
    bw = 1.06 * (_N ** -0.2) * jnp.mean(jnp.sqrt(var))
    proj2 = jnp.dot(mat.reshape(_E * _L * _N, _F), par_ref[...],
                    preferred_element_type=jnp.float32)
    proj_ref[...] = proj2
    left = jnp.min(proj2)
    right = jnp.max(proj2)
    h = (right - left) / (_S - 1)
    delta = (right - left) / _S
    c2 = (-0.5 / (bw * bw)) * _LOG2E
    divisor = jnp.sqrt(2.0 * jnp.pi) * bw
    scale = delta / (2.0 * divisor)
    vals = jnp.stack([left, h, c2, scale,
                      jnp.float32(0), jnp.float32(0),
                      jnp.float32(0), jnp.float32(0)])
    scal_ref[...] = jnp.broadcast_to(vals[:, None], (8, 128))


def _main_kernel(proj_ref, scal_ref, lenb_ref, acc_ref, kde_s):
    j = pl.program_id(0)

    @pl.when(j == 0)
    def _():
        acc_ref[...] = jnp.zeros_like(acc_ref)

    left = scal_ref[0, 0]
    h = scal_ref[1, 0]
    c2 = scal_ref[2, 0]
    lenb = lenb_ref[...]
    rlen = 1.0 / lenb
    ncorr = jnp.float32(_N) - lenb
    rlen_rows = [rlen[r:r + 1, :] for r in range(_E * _L)]
    ncorr_rows = [ncorr[r:r + 1, :] for r in range(_E * _L)]

    ones8 = jnp.ones((8, _F), jnp.float32)
    pv0 = (left + (j * _W).astype(jnp.float32) * h) * ones8
    hv = h * ones8
    c2v = c2 * ones8
    av = -2.0 * c2v * hv
    bv = c2v * hv * hv - av * pv0
    wv = jnp.exp2(2.0 * c2v * hv * hv)

    c2r = c2v[0:1, :]
    hr = hv[0:1, :]
    pj = pv0[0:1, :]
    cjs = []
    for _ in range(_W):
        cjs.append(jnp.exp2(c2r * (pj * pj)))
        pj = pj + hr

    ngrp = 4
    tiles_per_grp = _N // 8 // ngrp

    zeros_tail = jnp.zeros((8, _F), jnp.float32)
    for l in range(_L):
        for e in range(_E):
            el = e * _L + l
            row0 = el * _N

            def group(gi, accs, row0=row0):
                accs = list(accs)
                for t in range(tiles_per_grp):
                    r0 = row0 + (gi * tiles_per_grp + t) * 8
                    x = proj_ref[pl.ds(r0, 8), :]
                    d = x - pv0
                    g = jnp.exp2(c2v * (d * d))
                    r = jnp.exp2(av * x + bv)
                    for jj in range(_W - 1):
                        accs[jj] = accs[jj] + g
                        g = g * r
                        r = r * wv
                    accs[_W - 1] = accs[_W - 1] + g
                return tuple(accs)

            accs = lax.fori_loop(0, ngrp, group,
                                 (jnp.zeros((8, _F), jnp.float32),) * _W)
            kde_s[pl.ds(e * _JT + _JT - 8, 8), :] = zeros_tail
            for jj in range(_W):
                s = jnp.sum(accs[jj], axis=0, keepdims=True)
                kde = (s - ncorr_rows[el] * cjs[jj]) * rlen_rows[el]
                kde_s[pl.ds(e * _JT + jj, 1), :] = kde
        kd = [kde_s[pl.ds(e * _JT, _JT), :] for e in range(_E)]
        for k, (p, q) in enumerate(_PAIRS):
            acc_ref[k, l] += jnp.abs(kd[p] - kd[q])


def _final_kernel(acc_ref, scal_ref, tr_ref, te_ref, trd_ref, ted_ref):
    scale = scal_ref[3, 0]
    d = jnp.sum(acc_ref[...], axis=2) * scale
    test = jnp.max(d, axis=0)
    train = jnp.max(d[0:3], axis=0)
    tr_ref[...] = train
    te_ref[...] = test
    trd_ref[...] = jnp.mean(jnp.max(train, axis=0, keepdims=True),
                            axis=1, keepdims=True)
    ted_ref[...] = jnp.mean(jnp.max(test, axis=0, keepdims=True),
                            axis=1, keepdims=True)


def kernel(matrix, params, data_len):
    lens = data_len.astype(jnp.float32)
    lenb = jnp.broadcast_to(lens.reshape(_E * _L, 1), (_E * _L, _F))
    proj, scal = pl.pallas_call(
        _prep_kernel,
        out_shape=(
            jax.ShapeDtypeStruct((_E * _L * _N, _F), jnp.float32),
            jax.ShapeDtypeStruct((8, 128), jnp.float32),
        ),
        name="kde_prep",
    )(matrix, params)

    acc = pl.pallas_call(
        _main_kernel,
        grid=(_NSTEP,),
        in_specs=[
            pl.BlockSpec((_E * _L * _N, _F), lambda j: (0, 0)),
            pl.BlockSpec((8, 128), lambda j: (0, 0)),
            pl.BlockSpec((_E * _L, _F), lambda j: (0, 0)),
        ],
        out_specs=pl.BlockSpec((6, _L, _JT, _F), lambda j: (0, 0, 0, 0)),
        out_shape=jax.ShapeDtypeStruct((6, _L, _JT, _F), jnp.float32),
        scratch_shapes=[pltpu.VMEM((_E * _JT, _F), jnp.float32)],
        compiler_params=pltpu.CompilerParams(
            dimension_semantics=("arbitrary",),
        ),
        name="kde_main",
    )(proj, scal, lenb)

    tr, te, trd, ted = pl.pallas_call(
        _final_kernel,
        out_shape=(
            jax.ShapeDtypeStruct((_L, _F), jnp.float32),
            jax.ShapeDtypeStruct((_L, _F), jnp.float32),
            jax.ShapeDtypeStruct((1, 1), jnp.float32),
            jax.ShapeDtypeStruct((1, 1), jnp.float32),
        ),
        name="kde_final",
    )(acc, scal)
    return tr, te, trd[0, 0], ted[0, 0]


# speedup vs baseline: 1.0365x; 1.0365x over previous
import jax
import jax.numpy as jnp
from jax import lax
from jax.experimental import pallas as pl
from jax.experimental.pallas import tpu as pltpu

_E, _L, _N, _F = 4, 5, 512, 128
_S = 1000
_W = 25
_NSTEP = _S // _W
_JT = 32
_LOG2E = 1.4426950408889634
_PAIRS = ((0, 1), (0, 2), (1, 2), (0, 3), (1, 3), (2, 3))


def _prep_kernel(mat_ref, par_ref, proj_ref, scal_ref):
    mat = mat_ref[...]
    mean = jnp.mean(mat, axis=2, keepdims=True)
    var = jnp.sum((mat - mean) ** 2, axis=2) / (_N - 1)
    bw = 1.06 * (_N ** -0.2) * jnp.mean(jnp.sqrt(var))
    proj2 = jnp.dot(mat.reshape(_E * _L * _N, _F), par_ref[...],
                    preferred_element_type=jnp.float32)
    proj_ref[...] = proj2
    left = jnp.min(proj2)
    right = jnp.max(proj2)
    h = (right - left) / (_S - 1)
    delta = (right - left) / _S
    c2 = (-0.5 / (bw * bw)) * _LOG2E
    divisor = jnp.sqrt(2.0 * jnp.pi) * bw
    scale = delta / (2.0 * divisor)
    vals = jnp.stack([left, h, c2, scale,
                      jnp.float32(0), jnp.float32(0),
                      jnp.float32(0), jnp.float32(0)])
    scal_ref[...] = jnp.broadcast_to(vals[:, None], (8, 128))


def _main_kernel(proj_ref, scal_ref, lenb_ref, acc_ref, kde_s):
    j = pl.program_id(0)

    @pl.when(j == 0)
    def _():
        acc_ref[...] = jnp.zeros_like(acc_ref)

    left = scal_ref[0, 0]
    h = scal_ref[1, 0]
    c2 = scal_ref[2, 0]
    ones8 = jnp.ones((8, _F), jnp.float32)
    pv0 = (left + (j * _W).astype(jnp.float32) * h) * ones8
    hv = h * ones8
    c2v = c2 * ones8
    av = -2.0 * c2v * hv
    bv = c2v * hv * hv - av * pv0
    wv = jnp.exp2(2.0 * c2v * hv * hv)

    c2r = c2v[0:1, :]
    hr = hv[0:1, :]
    pj = pv0[0:1, :]
    cjs = []
    for _ in range(_W):
        cjs.append(jnp.exp2(c2r * (pj * pj)))
        pj = pj + hr

    ngrp = 2
    tiles_per_grp = _N // 8 // ngrp

    zeros_tail = jnp.zeros((1, 8, _F), jnp.float32)
    for l in range(_L):

        def env(e, _, l=l):
            row0 = (e * _L + l) * _N

            def group(gi, accs, row0=row0):
                accs = list(accs)
                for t in range(tiles_per_grp):
                    r0 = pl.multiple_of(row0 + (gi * tiles_per_grp + t) * 8,
                                        8)
                    x = proj_ref[pl.ds(r0, 8), :]
                    d = x - pv0
                    g = jnp.exp2(c2v * (d * d))
                    r = jnp.exp2(av * x + bv)
                    for jj in range(_W - 1):
                        accs[jj] = accs[jj] + g
                        g = g * r
                        r = r * wv
                    accs[_W - 1] = accs[_W - 1] + g
                return tuple(accs)

            accs = lax.fori_loop(0, ngrp, group,
                                 (jnp.zeros((8, _F), jnp.float32),) * _W)
            kde_s[pl.ds(e, 1), _JT - 8:, :] = zeros_tail
            lrow = lenb_ref[pl.ds(e * _L + l, 1), :, :][0]
            nc = jnp.float32(_N) - lrow
            rl = 1.0 / lrow
            for jj in range(_W):
                s = jnp.sum(accs[jj], axis=0, keepdims=True)
                kde = (s - nc * cjs[jj]) * rl
                kde_s[pl.ds(e, 1), jj, :] = kde
            return 0

        lax.fori_loop(0, _E, env, 0)
        kd = [kde_s[e] for e in range(_E)]
        for k, (p, q) in enumerate(_PAIRS):
            acc_ref[k, l] += jnp.abs(kd[p] - kd[q])


def _final_kernel(acc_ref, scal_ref, tr_ref, te_ref, trd_ref, ted_ref):
    scale = scal_ref[3, 0]
    d = jnp.sum(acc_ref[...], axis=2) * scale
    test = jnp.max(d, axis=0)
    train = jnp.max(d[0:3], axis=0)
    tr_ref[...] = train
    te_ref[...] = test
    trd_ref[...] = jnp.mean(jnp.max(train, axis=0, keepdims=True),
                            axis=1, keepdims=True)
    ted_ref[...] = jnp.mean(jnp.max(test, axis=0, keepdims=True),
                            axis=1, keepdims=True)


def kernel(matrix, params, data_len):
    lens = data_len.astype(jnp.float32)
    lenb = jnp.broadcast_to(lens.reshape(_E * _L, 1, 1), (_E * _L, 1, _F))
    proj, scal = pl.pallas_call(
        _prep_kernel,
        out_shape=(
            jax.ShapeDtypeStruct((_E * _L * _N, _F), jnp.float32),
            jax.ShapeDtypeStruct((8, 128), jnp.float32),
        ),
        name="kde_prep",
    )(matrix, params)

    acc = pl.pallas_call(
        _main_kernel,
        grid=(_NSTEP,),
        in_specs=[
            pl.BlockSpec((_E * _L * _N, _F), lambda j: (0, 0)),
            pl.BlockSpec((8, 128), lambda j: (0, 0)),
            pl.BlockSpec((_E * _L, 1, _F), lambda j: (0, 0, 0)),
        ],
        out_specs=pl.BlockSpec((6, _L, _JT, _F), lambda j: (0, 0, 0, 0)),
        out_shape=jax.ShapeDtypeStruct((6, _L, _JT, _F), jnp.float32),
        scratch_shapes=[pltpu.VMEM((_E, _JT, _F), jnp.float32)],
        compiler_params=pltpu.CompilerParams(
            dimension_semantics=("arbitrary",),
        ),
        name="kde_main",
    )(proj, scal, lenb)

    tr, te, trd, ted = pl.pallas_call(
        _final_kernel,
        out_shape=(
            jax.ShapeDtypeStruct((_L, _F), jnp.float32),
            jax.ShapeDtypeStruct((_L, _F), jnp.float32),
            jax.ShapeDtypeStruct((1, 1), jnp.float32),
            jax.ShapeDtypeStruct((1, 1), jnp.float32),
        ),
        name="kde_final",
    )(acc, scal)
    return tr, te, trd[0, 0], ted[0, 0]
